# baseline (speedup 1.0000x reference)
import functools

import jax
import jax.numpy as jnp
from jax import lax
from jax.experimental import pallas as pl
from jax.experimental.pallas import tpu as pltpu
from jax.experimental.pallas import tpu_sc as plsc

N_ROWS = 320000
D = 128
NUM_SEGMENTS = 10000
NC = 2
NS = 16
NW = NC * NS
SEG_PER_W = 320
S_PAD = NW * SEG_PER_W
CHUNK = 160
LANES = 16
NVREG = D // LANES
BOUNDS_PAD = 48
IDX_PAD = CHUNK + LANES


def _sc_body(proc_hbm, idx_hbm, bounds_hbm, m_hbm, s_hbm,
             bnd_v, rows_a, rows_b, idx_a, idx_bb, m_st, s_st,
             acc_m, acc_s, sem_ra, sem_rb, sem_ia, sem_ib):
    cid = lax.axis_index("c")
    sid = lax.axis_index("s")
    wid = sid * NC + cid
    seg_lo = pl.multiple_of(wid * SEG_PER_W, 8)

    neg_inf_v = jnp.full((LANES,), -jnp.inf, jnp.float32)
    zero_v = jnp.zeros((LANES,), jnp.float32)

    pltpu.sync_copy(bounds_hbm, bnd_v)
    bnd_vec = bnd_v[pl.ds(wid, LANES)]
    row_lo = bnd_vec[0]
    row_hi = bnd_vec[1]

    def init_body(i, _):
        for j in range(NVREG):
            m_st[pl.ds(i * D + j * LANES, LANES)] = neg_inf_v
            s_st[pl.ds(i * D + j * LANES, LANES)] = zero_v
        return 0
    lax.fori_loop(0, SEG_PER_W, init_body, 0)

    def load_acc():
        m = tuple(acc_m[pl.ds(j * LANES, LANES)] for j in range(NVREG))
        s = tuple(acc_s[pl.ds(j * LANES, LANES)] for j in range(NVREG))
        return m, s

    def store_acc(m, s):
        for j in range(NVREG):
            acc_m[pl.ds(j * LANES, LANES)] = m[j]
            acc_s[pl.ds(j * LANES, LANES)] = s[j]

    def init_acc():
        for j in range(NVREG):
            acc_m[pl.ds(j * LANES, LANES)] = neg_inf_v
            acc_s[pl.ds(j * LANES, LANES)] = zero_v

    def flush(g_cur):
        off = (g_cur - seg_lo) * D
        for j in range(NVREG):
            m_st[pl.ds(off + j * LANES, LANES)] = acc_m[pl.ds(j * LANES, LANES)]
            s_st[pl.ds(off + j * LANES, LANES)] = acc_s[pl.ds(j * LANES, LANES)]

    c0 = row_lo // CHUNK
    c1 = (row_hi + CHUNK - 1) // CHUNK

    def process(c, rows_v, idx_v, carry):
        base = pl.multiple_of(c * CHUNK, 8)
        pltpu.sync_copy(proc_hbm.at[pl.ds(base * D, CHUNK * D)], rows_v)
        pltpu.sync_copy(idx_hbm.at[pl.ds(base, CHUNK)],
                        idx_v.at[pl.ds(0, CHUNK)])
        i_lo = lax.max(row_lo - base, 0)
        i_hi = lax.min(row_hi - base, CHUNK)

        def row_body(i, g_cur):
            g = idx_v[pl.ds(i, LANES)][0]
            changed = g != g_cur

            @pl.when(jnp.logical_and(changed, g_cur >= 0))
            def _():
                flush(g_cur)

            @pl.when(changed)
            def _():
                init_acc()

            m, s = load_acc()
            new_m = []
            new_s = []
            for j in range(NVREG):
                x = rows_v[pl.ds(i * D + j * LANES, LANES)]
                m2 = jnp.maximum(m[j], x)
                s2 = s[j] * jnp.exp(m[j] - m2) + jnp.exp(x - m2)
                new_m.append(m2)
                new_s.append(s2)
            store_acc(tuple(new_m), tuple(new_s))
            return g

        return lax.fori_loop(i_lo, i_hi, row_body, carry)

    def chunk_body(c, carry):
        return process(c, rows_a, idx_a, carry)

    g_cur = lax.fori_loop(c0, c1, chunk_body, jnp.int32(-1))

    @pl.when(g_cur >= 0)
    def _():
        flush(g_cur)

    out_off = pl.multiple_of(seg_lo * D, 8)
    pltpu.sync_copy(m_st, m_hbm.at[pl.ds(out_off, SEG_PER_W * D)])
    pltpu.sync_copy(s_st, s_hbm.at[pl.ds(out_off, SEG_PER_W * D)])


_sc_call = functools.partial(
    pl.kernel,
    out_type=(
        jax.ShapeDtypeStruct((S_PAD * D,), jnp.float32),
        jax.ShapeDtypeStruct((S_PAD * D,), jnp.float32),
    ),
    mesh=plsc.VectorSubcoreMesh(
        core_axis_name="c", subcore_axis_name="s",
        num_cores=NC, num_subcores=NS,
    ),
    scratch_types=[
        pltpu.VMEM((BOUNDS_PAD,), jnp.int32),
        pltpu.VMEM((CHUNK * D,), jnp.float32),
        pltpu.VMEM((CHUNK * D,), jnp.float32),
        pltpu.VMEM((IDX_PAD,), jnp.int32),
        pltpu.VMEM((IDX_PAD,), jnp.int32),
        pltpu.VMEM((SEG_PER_W * D,), jnp.float32),
        pltpu.VMEM((SEG_PER_W * D,), jnp.float32),
        pltpu.VMEM((D,), jnp.float32),
        pltpu.VMEM((D,), jnp.float32),
        pltpu.SemaphoreType.DMA,
        pltpu.SemaphoreType.DMA,
        pltpu.SemaphoreType.DMA,
        pltpu.SemaphoreType.DMA,
    ],
)(_sc_body)


def _finalize_body(m_ref, s_ref, out_ref):
    m = m_ref[0:NUM_SEGMENTS, :]
    s = s_ref[0:NUM_SEGMENTS, :]
    out = jnp.log(s) + m
    gmax = jnp.max(out)
    t = jnp.sum(jnp.exp(out - gmax))
    z = jnp.log(t) + gmax
    out_ref[...] = out - z


_finalize_call = pl.pallas_call(
    _finalize_body,
    out_shape=jax.ShapeDtypeStruct((NUM_SEGMENTS, D), jnp.float32),
)


@jax.jit
def kernel(proc, idx_b):
    seg_starts = jnp.arange(NW + 1, dtype=jnp.int32) * SEG_PER_W
    bounds = jnp.searchsorted(idx_b, seg_starts, side="left").astype(jnp.int32)
    bounds = jnp.pad(bounds, (0, BOUNDS_PAD - (NW + 1)))
    m_all, s_all = _sc_call(proc.reshape(N_ROWS * D), idx_b, bounds)
    return _finalize_call(m_all.reshape(S_PAD, D), s_all.reshape(S_PAD, D))


# speedup vs baseline: 1.0960x; 1.0960x over previous
import functools

import jax
import jax.numpy as jnp
from jax import lax
from jax.experimental import pallas as pl
from jax.experimental.pallas import tpu as pltpu
from jax.experimental.pallas import tpu_sc as plsc

N_ROWS = 320000
D = 128
NUM_SEGMENTS = 10000
NC = 2
NS = 16
NW = NC * NS
SEG_PER_W = 320
S_PAD = NW * SEG_PER_W
CHUNK = 160
N_CHUNKS = N_ROWS // CHUNK
LANES = 16
NVREG = D // LANES
ST_LOCAL = SEG_PER_W + LANES
ST_GLOBAL = (NW - 1) * SEG_PER_W + ST_LOCAL
LAST_PAD = N_CHUNKS + LANES


def _sc_body(proc_hbm, starts_hbm, last_hbm, m_hbm, s_hbm,
             st_v, last_v, rows_a, rows_b, m_st, s_st, acc_m, acc_s,
             sem_ra, sem_rb, sem_st, sem_la):
    cid = lax.axis_index("c")
    sid = lax.axis_index("s")
    wid = sid * NC + cid
    seg_lo = pl.multiple_of(wid * SEG_PER_W, 8)

    neg_inf_v = jnp.full((LANES,), -jnp.inf, jnp.float32)
    zero_v = jnp.zeros((LANES,), jnp.float32)

    pltpu.make_async_copy(
        starts_hbm.at[pl.ds(seg_lo, ST_LOCAL)], st_v, sem_st).start()
    pltpu.make_async_copy(last_hbm, last_v, sem_la).start()

    def init_body(i, _):
        for j in range(NVREG):
            m_st[pl.ds(i * D + j * LANES, LANES)] = neg_inf_v
            s_st[pl.ds(i * D + j * LANES, LANES)] = zero_v
        return 0
    lax.fori_loop(0, SEG_PER_W, init_body, 0)

    def reset_acc():
        for j in range(NVREG):
            acc_m[pl.ds(j * LANES, LANES)] = neg_inf_v
            acc_s[pl.ds(j * LANES, LANES)] = zero_v
    reset_acc()

    pltpu.make_async_copy(
        starts_hbm.at[pl.ds(seg_lo, ST_LOCAL)], st_v, sem_st).wait()
    pltpu.make_async_copy(last_hbm, last_v, sem_la).wait()

    def st_at(k):
        return st_v[pl.ds(k, LANES)][0]

    row_lo = st_at(0)
    row_hi = st_at(SEG_PER_W)
    seg_hi = seg_lo + SEG_PER_W

    def flush(p):
        off = (p - seg_lo) * D
        for j in range(NVREG):
            jo = j * LANES
            m_st[pl.ds(off + jo, LANES)] = acc_m[pl.ds(jo, LANES)]
            s_st[pl.ds(off + jo, LANES)] = acc_s[pl.ds(jo, LANES)]

    def row_dma(c, rows_buf, sem):
        base = pl.multiple_of(c * CHUNK, 8)
        return pltpu.make_async_copy(
            proc_hbm.at[pl.ds(base * D, CHUNK * D)], rows_buf, sem)

    c0 = row_lo // CHUNK
    c1 = (row_hi + CHUNK - 1) // CHUNK

    @pl.when(c1 > c0)
    def _():
        row_dma(c0, rows_a, sem_ra).start()

    def process(c, rows_v, sem, n_rows, n_sem, p_in):
        row_dma(c, rows_v, sem).wait()

        @pl.when(c + 1 < c1)
        def _():
            row_dma(c + 1, n_rows, n_sem).start()

        base = pl.multiple_of(c * CHUNK, 8)
        i_lo = lax.max(row_lo - base, 0)
        i_hi = lax.min(row_hi - base, CHUNK)
        end = base + i_hi

        def quad(r0, _):
            ro = r0 * D
            for j in range(NVREG):
                o = ro + j * LANES
                jo = j * LANES
                m = acc_m[pl.ds(jo, LANES)]
                s = acc_s[pl.ds(jo, LANES)]
                x0 = rows_v[pl.ds(o, LANES)]
                x1 = rows_v[pl.ds(o + D, LANES)]
                x2 = rows_v[pl.ds(o + 2 * D, LANES)]
                x3 = rows_v[pl.ds(o + 3 * D, LANES)]
                mx = jnp.maximum(jnp.maximum(x0, x1), jnp.maximum(x2, x3))
                m2 = jnp.maximum(m, mx)
                e = (jnp.exp(x0 - m2) + jnp.exp(x1 - m2)) + \
                    (jnp.exp(x2 - m2) + jnp.exp(x3 - m2))
                s2 = s * jnp.exp(m - m2) + e
                acc_m[pl.ds(jo, LANES)] = m2
                acc_s[pl.ds(jo, LANES)] = s2
            return 0

        def single(r, _):
            for j in range(NVREG):
                jo = j * LANES
                m = acc_m[pl.ds(jo, LANES)]
                s = acc_s[pl.ds(jo, LANES)]
                x = rows_v[pl.ds(r * D + jo, LANES)]
                m2 = jnp.maximum(m, x)
                s2 = s * jnp.exp(m - m2) + jnp.exp(x - m2)
                acc_m[pl.ds(jo, LANES)] = m2
                acc_s[pl.ds(jo, LANES)] = s2
            return 0

        p_hi = lax.min(last_v[pl.ds(c, LANES)][0] + 1, seg_hi)

        def seg_body(p, _):
            e1 = st_at(p - seg_lo + 1)
            r0 = lax.max(st_at(p - seg_lo) - base, i_lo)
            r1 = lax.min(e1 - base, i_hi)
            nq = (r1 - r0) // 4
            lax.fori_loop(0, nq, lambda q, cc: quad(r0 + q * 4, cc), 0)
            lax.fori_loop(r0 + nq * 4, r1, single, 0)

            @pl.when(e1 <= end)
            def _():
                flush(p)
                reset_acc()
            return 0

        lax.fori_loop(p_in, p_hi, seg_body, 0)
        last_done = st_at(p_hi - seg_lo) <= end
        return jnp.where(last_done, p_hi, p_hi - 1)

    def chunk_body(c, p):
        even = ((c - c0) % 2) == 0
        return lax.cond(
            even,
            lambda pp: process(c, rows_a, sem_ra, rows_b, sem_rb, pp),
            lambda pp: process(c, rows_b, sem_rb, rows_a, sem_ra, pp),
            p)

    lax.fori_loop(c0, c1, chunk_body, jnp.int32(0) + seg_lo)

    out_off = pl.multiple_of(seg_lo * D, 8)
    pltpu.sync_copy(m_st, m_hbm.at[pl.ds(out_off, SEG_PER_W * D)])
    pltpu.sync_copy(s_st, s_hbm.at[pl.ds(out_off, SEG_PER_W * D)])


_sc_call = functools.partial(
    pl.kernel,
    out_type=(
        jax.ShapeDtypeStruct((S_PAD * D,), jnp.float32),
        jax.ShapeDtypeStruct((S_PAD * D,), jnp.float32),
    ),
    mesh=plsc.VectorSubcoreMesh(
        core_axis_name="c", subcore_axis_name="s",
        num_cores=NC, num_subcores=NS,
    ),
    scratch_types=[
        pltpu.VMEM((ST_LOCAL,), jnp.int32),
        pltpu.VMEM((LAST_PAD,), jnp.int32),
        pltpu.VMEM((CHUNK * D,), jnp.float32),
        pltpu.VMEM((CHUNK * D,), jnp.float32),
        pltpu.VMEM((SEG_PER_W * D,), jnp.float32),
        pltpu.VMEM((SEG_PER_W * D,), jnp.float32),
        pltpu.VMEM((D,), jnp.float32),
        pltpu.VMEM((D,), jnp.float32),
        pltpu.SemaphoreType.DMA,
        pltpu.SemaphoreType.DMA,
        pltpu.SemaphoreType.DMA,
        pltpu.SemaphoreType.DMA,
    ],
)(_sc_body)


def _finalize_body(m_ref, s_ref, out_ref):
    m = m_ref[0:NUM_SEGMENTS, :]
    s = s_ref[0:NUM_SEGMENTS, :]
    out = jnp.log(s) + m
    gmax = jnp.max(out)
    t = jnp.sum(jnp.exp(out - gmax))
    z = jnp.log(t) + gmax
    out_ref[...] = out - z


_finalize_call = pl.pallas_call(
    _finalize_body,
    out_shape=jax.ShapeDtypeStruct((NUM_SEGMENTS, D), jnp.float32),
)


@jax.jit
def kernel(proc, idx_b):
    qs = jnp.arange(ST_GLOBAL, dtype=jnp.int32)
    starts = jnp.searchsorted(idx_b, qs, side="left").astype(jnp.int32)
    last = jnp.pad(idx_b.reshape(N_CHUNKS, CHUNK)[:, CHUNK - 1],
                   (0, LAST_PAD - N_CHUNKS), constant_values=NUM_SEGMENTS)
    m_all, s_all = _sc_call(proc.reshape(N_ROWS * D), starts, last)
    return _finalize_call(m_all.reshape(S_PAD, D), s_all.reshape(S_PAD, D))


# speedup vs baseline: 2.1555x; 1.9666x over previous
import functools

import jax
import jax.numpy as jnp
from jax import lax
from jax.experimental import pallas as pl
from jax.experimental.pallas import tpu as pltpu
from jax.experimental.pallas import tpu_sc as plsc

N_ROWS = 320000
D = 128
NUM_SEGMENTS = 10000
NC = 2
NS = 16
NW = NC * NS
SEG_PER_W = 320
S_PAD = NW * SEG_PER_W
CHUNK = 160
N_CHUNKS = N_ROWS // CHUNK
LANES = 16
NVREG = D // LANES
NGROUPS = CHUNK // LANES
BOUNDS_PAD = 48
LAST_PAD = N_CHUNKS + LANES


def _sc_body(proc_hbm, idx_hbm, bounds_hbm, last_hbm, m_hbm, s_hbm,
             bnd_v, last_v, rows_a, rows_b, idx_va, idx_vb, m_st, s_st,
             acc_m, acc_s, sem_ra, sem_rb, sem_ia, sem_ib):
    cid = lax.axis_index("c")
    sid = lax.axis_index("s")
    wid = sid * NC + cid
    seg_lo = pl.multiple_of(wid * SEG_PER_W, 8)
    seg_hi = seg_lo + SEG_PER_W

    neg_inf_v = jnp.full((LANES,), -jnp.inf, jnp.float32)
    zero_v = jnp.zeros((LANES,), jnp.float32)

    pltpu.sync_copy(bounds_hbm, bnd_v)
    pltpu.sync_copy(last_hbm, last_v)

    bnd_vec = bnd_v[pl.ds(wid, LANES)]
    row_lo = bnd_vec[0]
    row_hi = bnd_vec[1]

    def init_body(i, _):
        for j in range(NVREG):
            m_st[pl.ds(i * D + j * LANES, LANES)] = neg_inf_v
            s_st[pl.ds(i * D + j * LANES, LANES)] = zero_v
        return 0
    lax.fori_loop(0, SEG_PER_W, init_body, 0)

    def reset_acc():
        for j in range(NVREG):
            acc_m[pl.ds(j * LANES, LANES)] = neg_inf_v
            acc_s[pl.ds(j * LANES, LANES)] = zero_v
    reset_acc()

    def flush(p):
        off = (p - seg_lo) * D
        for j in range(NVREG):
            jo = j * LANES
            m_st[pl.ds(off + jo, LANES)] = acc_m[pl.ds(jo, LANES)]
            s_st[pl.ds(off + jo, LANES)] = acc_s[pl.ds(jo, LANES)]

    def dma_handles(c, rows_buf, idx_buf, semr, semi):
        base = pl.multiple_of(c * CHUNK, 8)
        hr = pltpu.make_async_copy(
            proc_hbm.at[pl.ds(base * D, CHUNK * D)], rows_buf, semr)
        hi = pltpu.make_async_copy(
            idx_hbm.at[pl.ds(base, CHUNK)], idx_buf.at[pl.ds(0, CHUNK)], semi)
        return hr, hi

    def start_dma(c, rows_buf, idx_buf, semr, semi):
        hr, hi = dma_handles(c, rows_buf, idx_buf, semr, semi)
        hr.start()
        hi.start()

    c0 = row_lo // CHUNK
    c1 = (row_hi + CHUNK - 1) // CHUNK

    @pl.when(c1 > c0)
    def _():
        start_dma(c0, rows_a, idx_va, sem_ra, sem_ia)

    def process(c, rows_v, idx_v, semr, semi, n_rows, n_idx, n_semr, n_semi,
                p_in):
        hr, hi = dma_handles(c, rows_v, idx_v, semr, semi)
        hr.wait()
        hi.wait()

        @pl.when(c + 1 < c1)
        def _():
            start_dma(c + 1, n_rows, n_idx, n_semr, n_semi)

        base = pl.multiple_of(c * CHUNK, 8)
        i_lo = lax.max(row_lo - base, 0)
        i_hi = lax.min(row_hi - base, CHUNK)
        end = base + i_hi
        final = end == row_hi

        def octo(r0, _):
            ro = r0 * D
            for j in range(NVREG):
                o = ro + j * LANES
                jo = j * LANES
                m = acc_m[pl.ds(jo, LANES)]
                s = acc_s[pl.ds(jo, LANES)]
                x0 = rows_v[pl.ds(o, LANES)]
                x1 = rows_v[pl.ds(o + D, LANES)]
                x2 = rows_v[pl.ds(o + 2 * D, LANES)]
                x3 = rows_v[pl.ds(o + 3 * D, LANES)]
                x4 = rows_v[pl.ds(o + 4 * D, LANES)]
                x5 = rows_v[pl.ds(o + 5 * D, LANES)]
                x6 = rows_v[pl.ds(o + 6 * D, LANES)]
                x7 = rows_v[pl.ds(o + 7 * D, LANES)]
                mx = jnp.maximum(
                    jnp.maximum(jnp.maximum(x0, x1), jnp.maximum(x2, x3)),
                    jnp.maximum(jnp.maximum(x4, x5), jnp.maximum(x6, x7)))
                m2 = jnp.maximum(m, mx)
                e = ((jnp.exp(x0 - m2) + jnp.exp(x1 - m2)) +
                     (jnp.exp(x2 - m2) + jnp.exp(x3 - m2))) + \
                    ((jnp.exp(x4 - m2) + jnp.exp(x5 - m2)) +
                     (jnp.exp(x6 - m2) + jnp.exp(x7 - m2)))
                s2 = s * jnp.exp(m - m2) + e
                acc_m[pl.ds(jo, LANES)] = m2
                acc_s[pl.ds(jo, LANES)] = s2
            return 0

        def single(r, _):
            for j in range(NVREG):
                jo = j * LANES
                m = acc_m[pl.ds(jo, LANES)]
                s = acc_s[pl.ds(jo, LANES)]
                x = rows_v[pl.ds(r * D + jo, LANES)]
                m2 = jnp.maximum(m, x)
                s2 = s * jnp.exp(m - m2) + jnp.exp(x - m2)
                acc_m[pl.ds(jo, LANES)] = m2
                acc_s[pl.ds(jo, LANES)] = s2
            return 0

        p_hi = lax.min(last_v[pl.ds(c, LANES)][0] + 1, seg_hi)

        def seg_body(p, i):
            def bs_step(_, lh):
                lo, hi = lh
                mid = (lo + hi) // 2
                v = idx_v[pl.ds(mid, LANES)][0]
                gt = v > p
                return jnp.where(gt, lo, mid + 1), jnp.where(gt, mid, hi)
            r1, _ = lax.fori_loop(0, 8, bs_step, (i, i_hi))
            nb = (r1 - i) // 8
            lax.fori_loop(0, nb, lambda b, cc: octo(i + b * 8, cc), 0)
            lax.fori_loop(i + nb * 8, r1, single, 0)
            done = jnp.logical_or(r1 < i_hi, final)

            @pl.when(done)
            def _():
                flush(p)
                reset_acc()
            return r1

        cursor = lax.fori_loop(p_in, p_hi, seg_body, i_lo)
        last_done = jnp.logical_or(cursor < i_hi, final)
        return jnp.where(last_done, p_hi, p_hi - 1)

    def chunk_body(c, p):
        even = ((c - c0) % 2) == 0
        return lax.cond(
            even,
            lambda pp: process(c, rows_a, idx_va, sem_ra, sem_ia,
                               rows_b, idx_vb, sem_rb, sem_ib, pp),
            lambda pp: process(c, rows_b, idx_vb, sem_rb, sem_ib,
                               rows_a, idx_va, sem_ra, sem_ia, pp),
            p)

    lax.fori_loop(c0, c1, chunk_body, jnp.int32(0) + seg_lo)

    out_off = pl.multiple_of(seg_lo * D, 8)
    pltpu.sync_copy(m_st, m_hbm.at[pl.ds(out_off, SEG_PER_W * D)])
    pltpu.sync_copy(s_st, s_hbm.at[pl.ds(out_off, SEG_PER_W * D)])


_sc_call = functools.partial(
    pl.kernel,
    out_type=(
        jax.ShapeDtypeStruct((S_PAD * D,), jnp.float32),
        jax.ShapeDtypeStruct((S_PAD * D,), jnp.float32),
    ),
    mesh=plsc.VectorSubcoreMesh(
        core_axis_name="c", subcore_axis_name="s",
        num_cores=NC, num_subcores=NS,
    ),
    scratch_types=[
        pltpu.VMEM((BOUNDS_PAD,), jnp.int32),
        pltpu.VMEM((LAST_PAD,), jnp.int32),
        pltpu.VMEM((CHUNK * D,), jnp.float32),
        pltpu.VMEM((CHUNK * D,), jnp.float32),
        pltpu.VMEM((CHUNK + LANES,), jnp.int32),
        pltpu.VMEM((CHUNK + LANES,), jnp.int32),
        pltpu.VMEM((SEG_PER_W * D,), jnp.float32),
        pltpu.VMEM((SEG_PER_W * D,), jnp.float32),
        pltpu.VMEM((D,), jnp.float32),
        pltpu.VMEM((D,), jnp.float32),
        pltpu.SemaphoreType.DMA,
        pltpu.SemaphoreType.DMA,
        pltpu.SemaphoreType.DMA,
        pltpu.SemaphoreType.DMA,
    ],
)(_sc_body)


def _finalize_body(m_ref, s_ref, out_ref):
    m = m_ref[0:NUM_SEGMENTS, :]
    s = s_ref[0:NUM_SEGMENTS, :]
    out = jnp.log(s) + m
    gmax = jnp.max(out)
    t = jnp.sum(jnp.exp(out - gmax))
    z = jnp.log(t) + gmax
    out_ref[...] = out - z


_finalize_call = pl.pallas_call(
    _finalize_body,
    out_shape=jax.ShapeDtypeStruct((NUM_SEGMENTS, D), jnp.float32),
)


@jax.jit
def kernel(proc, idx_b):
    seg_starts = jnp.arange(NW + 1, dtype=jnp.int32) * SEG_PER_W
    bounds = jnp.searchsorted(idx_b, seg_starts, side="left").astype(jnp.int32)
    bounds = jnp.pad(bounds, (0, BOUNDS_PAD - (NW + 1)))
    last = jnp.pad(idx_b.reshape(N_CHUNKS, CHUNK)[:, CHUNK - 1],
                   (0, LAST_PAD - N_CHUNKS), constant_values=NUM_SEGMENTS)
    m_all, s_all = _sc_call(proc.reshape(N_ROWS * D), idx_b, bounds, last)
    return _finalize_call(m_all.reshape(S_PAD, D), s_all.reshape(S_PAD, D))
